# baseline (speedup 1.0000x reference)
import functools

import jax
import jax.numpy as jnp
from jax import lax
from jax.experimental import pallas as pl
from jax.experimental.pallas import tpu as pltpu
from jax.experimental.pallas import tpu_sc as plsc

_D = 64
_BT = 128
_NBUF = 4
_NC = 2
_NS = 16
_NW = _NC * _NS
_L = 16


def _k1_body(tin_ref, tout_ref):
    blk = tin_ref[...]
    t = blk.T
    t4 = t.reshape(256, 2, 64)
    tout_ref[...] = jnp.concatenate([t4[:, 0, :], t4[:, 1, :]], axis=1)


def _table_rowmajor(tT):
    return pl.pallas_call(
        _k1_body,
        grid=(1954,),
        in_specs=[pl.BlockSpec((64, 512), lambda i: (0, i))],
        out_specs=pl.BlockSpec((256, 128), lambda i: (i, 0)),
        out_shape=jax.ShapeDtypeStruct((500000, 128), jnp.float32),
    )(tT)


def _k2_body(xT_hbm, tbl_hbm, out_hbm, idx_all, *rest, hist):
    gbufs = rest[:_NBUF]
    tbufs = rest[_NBUF:2 * _NBUF]
    gsems = rest[2 * _NBUF:3 * _NBUF]
    ssems = rest[3 * _NBUF:4 * _NBUF]

    wid = lax.axis_index("s") * _NC + lax.axis_index("c")

    pltpu.sync_copy(xT_hbm.at[:, pl.ds(wid * _BT, _BT)], idx_all)

    def fire_gather(b, h):
        pltpu.async_copy(tbl_hbm.at[idx_all.at[h]], gbufs[b], gsems[b])

    def wait_gather(b, h):
        pltpu.make_async_copy(tbl_hbm.at[idx_all.at[h]], gbufs[b],
                              gsems[b]).wait()

    def fire_store(b, h):
        pltpu.async_copy(tbufs[b], out_hbm.at[h, :, wid], ssems[b])

    def wait_store(b, h):
        pltpu.make_async_copy(tbufs[b], out_hbm.at[h, :, wid],
                              ssems[b]).wait()

    def transpose_fix(b, h):
        gbuf, tbuf = gbufs[b], tbufs[b]
        masks = []
        for k in range(_BT // _L):
            ivec = idx_all[h, pl.ds(k * _L, _L)]
            masks.append(ivec == 0)
        zeros = jnp.zeros((_L,), jnp.float32)
        base = lax.iota(jnp.int32, _L)

        def dloop(d, carry):
            dt = d // 8
            ds = d % 8
            col = jnp.zeros((_L,), jnp.int32) + d
            for k in range(_BT // _L):
                v = plsc.load_gather(gbuf, [base + k * _L, col])
                v = jnp.where(masks[k], zeros, v)
                tbuf[dt, ds, pl.ds(k * _L, _L)] = v
            return carry

        lax.fori_loop(0, _D, dloop, 0)

    for b in range(_NBUF):
        fire_gather(b, b)

    def outer(i, carry):
        for b in range(_NBUF):
            h = i * _NBUF + b
            wait_gather(b, h)
            transpose_fix(b, h)
            fire_store(b, h)

            @pl.when(h + _NBUF < hist)
            def _():
                wait_store(b, h)
                fire_gather(b, h + _NBUF)
        return carry

    lax.fori_loop(0, hist // _NBUF, outer, 0)

    for b in range(_NBUF):
        wait_store(b, hist - _NBUF + b)


def kernel(X, table):
    batch, hist = X.shape
    vocab = table.shape[0]
    n_bt = batch // _BT

    tT = table.T
    tbl = _table_rowmajor(tT).reshape(vocab, _D)
    xT = X.T

    mesh = plsc.VectorSubcoreMesh(core_axis_name="c", subcore_axis_name="s",
                                  num_cores=_NC, num_subcores=_NS)
    scratch = (
        [pltpu.VMEM((hist, _BT), jnp.int32)]
        + [pltpu.VMEM((_BT, _D), jnp.float32)] * _NBUF
        + [pltpu.VMEM((8, 8, _BT), jnp.float32)] * _NBUF
        + [pltpu.SemaphoreType.DMA] * (2 * _NBUF)
    )
    out5 = pl.kernel(
        functools.partial(_k2_body, hist=hist),
        out_type=jax.ShapeDtypeStruct((hist, 8, n_bt, 8, _BT), jnp.float32),
        mesh=mesh,
        scratch_types=scratch,
        compiler_params=pltpu.CompilerParams(needs_layout_passes=False,
                                             use_tc_tiling_on_sc=False),
    )(xT, tbl)
    return out5.transpose(2, 4, 0, 1, 3).reshape(batch, hist, _D)
